# baseline (speedup 1.0000x reference)
import jax
import jax.numpy as jnp
from jax import lax
from jax.experimental import pallas as pl
from jax.experimental.pallas import tpu as pltpu

_N = 2048
_D = 3
_G = 0.01
_EPS = 1e-06
_BD = 1024


def _nbody_body(x_ref, m_ref, o_ref, xT_s):
    i = pl.program_id(0)

    @pl.when(i == 0)
    def _build_transposed():
        xT_s[0:6, :] = jnp.transpose(x_ref[:, 0:6], (1, 0))
        xT_s[6:7, :] = jnp.transpose(m_ref[:, 0:1], (1, 0))

    base = i * _BD
    qxd = x_ref[pl.ds(base, _BD), 0:1]
    qyd = x_ref[pl.ds(base, _BD), 1:2]
    qzd = x_ref[pl.ds(base, _BD), 2:3]
    qxs = xT_s[0:1, :]
    qys = xT_s[1:2, :]
    qzs = xT_s[2:3, :]

    dx = qxd - qxs
    dy = qyd - qys
    dz = qzd - qzs
    r2 = dx * dx + dy * dy + dz * dz
    u = lax.rsqrt(jnp.maximum(r2, 1e-30))
    s = r2 * u
    e = s + _EPS
    w = xT_s[6:7, :] * lax.reciprocal(e * e * e)

    fx = jnp.sum(dx * w, axis=1, keepdims=True)
    fy = jnp.sum(dy * w, axis=1, keepdims=True)
    fz = jnp.sum(dz * w, axis=1, keepdims=True)

    mj = m_ref[pl.ds(base, _BD), 0:1]
    scale = -_G * mj
    dotq = x_ref[pl.ds(base, _BD), 3:6] / mj
    o_ref[:, 0:3] = dotq
    o_ref[:, 3:6] = jnp.concatenate([fx * scale, fy * scale, fz * scale], axis=1)


def kernel(t, x, m):
    del t
    out = pl.pallas_call(
        _nbody_body,
        grid=(_N // _BD,),
        in_specs=[
            pl.BlockSpec((_N, 2 * _D), lambda i: (0, 0)),
            pl.BlockSpec((_N, 1), lambda i: (0, 0)),
        ],
        out_specs=pl.BlockSpec((_BD, 2 * _D), lambda i: (i, 0)),
        out_shape=jax.ShapeDtypeStruct((_N, 2 * _D), jnp.float32),
        scratch_shapes=[pltpu.VMEM((8, _N), jnp.float32)],
        compiler_params=pltpu.CompilerParams(
            vmem_limit_bytes=100 * 1024 * 1024,
        ),
    )(x, m)
    return out
